# baseline (speedup 1.0000x reference)
import functools

import jax
import jax.numpy as jnp
from jax import lax
from jax.experimental import pallas as pl
from jax.experimental.pallas import tpu as pltpu
from jax.experimental.pallas import tpu_sc as plsc

_NC = 2
_NS = 16
_NW = _NC * _NS

_WIN = 32
_L = 16


def _sc_gather(n_per_worker, n_windows, num_sample, feat_dim, b_pad):
    mesh = plsc.VectorSubcoreMesh(core_axis_name="c", subcore_axis_name="s")
    rowsz = feat_dim // 2
    nbuf_sz = _WIN * num_sample * rowsz
    obuf_sz = _WIN * rowsz

    assert n_windows % 2 == 1 and n_windows >= 3

    @functools.partial(
        pl.kernel,
        out_type=(
            jax.ShapeDtypeStruct((b_pad * rowsz,), jnp.int32),
            jax.ShapeDtypeStruct((b_pad * rowsz,), jnp.int32),
        ),
        mesh=mesh,
        scratch_types=[
            pltpu.VMEM((_WIN,), jnp.int32),
            pltpu.VMEM((_WIN,), jnp.int32),
            pltpu.VMEM((num_sample * _WIN,), jnp.int32),
            pltpu.VMEM((num_sample * _WIN,), jnp.int32),
            pltpu.VMEM((nbuf_sz,), jnp.int32),
            pltpu.VMEM((nbuf_sz,), jnp.int32),
            pltpu.VMEM((obuf_sz,), jnp.int32),
            pltpu.VMEM((obuf_sz,), jnp.int32),
            pltpu.VMEM((obuf_sz,), jnp.int32),
            pltpu.SemaphoreType.DMA,
            pltpu.SemaphoreType.DMA,
            pltpu.SemaphoreType.DMA,
        ],
    )
    def sc_kernel(nodes_hbm, narr_hbm, tab_hbm, self_hbm, sum_hbm,
                  nod0, nod1, nid0, nid1, rows0, rows1, self0, self1, acc_v,
                  sem_a, sem_b, sem_i):
        wid = lax.axis_index("s") * _NC + lax.axis_index("c")
        base = wid * n_per_worker

        def fire_idx(w, nod_iv, nid_iv):
            pltpu.async_copy(
                nodes_hbm.at[pl.ds(base + w * _WIN, _WIN)], nod_iv, sem_i)
            pltpu.async_copy(
                narr_hbm.at[pl.ds((wid * n_windows + w) * num_sample * _WIN,
                                  num_sample * _WIN)],
                nid_iv, sem_i)

        def wait_idx(nod_iv, nid_iv):
            pltpu.make_async_copy(
                nodes_hbm.at[pl.ds(0, _WIN)], nod_iv, sem_i).wait()
            pltpu.make_async_copy(
                narr_hbm.at[pl.ds(0, num_sample * _WIN)], nid_iv, sem_i).wait()

        def fire_gather(nod_iv, nid_iv, nrows_v, self_v, sem):
            def group_fire(g, carry):
                nv = nod_iv[pl.ds(g * _L, _L)]
                for i in range(_L):
                    s = nv[i] * rowsz
                    pltpu.async_copy(
                        tab_hbm.at[pl.ds(s, rowsz)],
                        self_v.at[pl.ds((g * _L + i) * rowsz, rowsz)], sem)
                for j in range(num_sample):
                    tv = nid_iv[pl.ds(j * _WIN + g * _L, _L)]
                    for i in range(_L):
                        t = tv[i] * rowsz
                        pltpu.async_copy(
                            tab_hbm.at[pl.ds(t, rowsz)],
                            nrows_v.at[pl.ds(
                                ((g * _L + i) * num_sample + j) * rowsz,
                                rowsz)],
                            sem)
                return carry

            lax.fori_loop(0, _WIN // _L, group_fire, 0)

        def drain_gather(nrows_v, self_v, sem):
            pltpu.make_async_copy(
                tab_hbm.at[pl.ds(0, nbuf_sz)], nrows_v, sem).wait()
            pltpu.make_async_copy(
                tab_hbm.at[pl.ds(0, obuf_sz)], self_v, sem).wait()

        def reduce_write(w, nrows_v, self_v):
            def node_red(i, carry):
                noff = i * num_sample * rowsz
                for c in range(rowsz // _L):
                    off = noff + c * _L
                    w = nrows_v[pl.ds(off, _L)]
                    lo = lax.bitcast_convert_type(w << 16, jnp.float32)
                    hi = lax.bitcast_convert_type(w & -65536, jnp.float32)
                    for j in range(1, num_sample):
                        w = nrows_v[pl.ds(off + j * rowsz, _L)]
                        lo = lo + lax.bitcast_convert_type(w << 16, jnp.float32)
                        hi = hi + lax.bitcast_convert_type(w & -65536, jnp.float32)
                    lo_w = lax.shift_right_logical(
                        lax.bitcast_convert_type(lo, jnp.int32) + 32768, 16)
                    hi_w = (lax.bitcast_convert_type(hi, jnp.int32) + 32768) & -65536
                    acc_v[pl.ds(i * rowsz + c * _L, _L)] = lo_w | hi_w
                return carry

            lax.fori_loop(0, _WIN, node_red, 0)
            row0 = base + w * _WIN
            pltpu.sync_copy(self_v, self_hbm.at[pl.ds(row0 * rowsz, obuf_sz)])
            pltpu.sync_copy(acc_v, sum_hbm.at[pl.ds(row0 * rowsz, obuf_sz)])

        fire_idx(0, nod0, nid0)
        wait_idx(nod0, nid0)
        fire_gather(nod0, nid0, rows0, self0, sem_a)
        fire_idx(1, nod1, nid1)

        def pair_body(p, carry):
            w0 = 2 * p
            w3 = w0 + 3
            wait_idx(nod1, nid1)
            fire_gather(nod1, nid1, rows1, self1, sem_b)
            fire_idx(w0 + 2, nod0, nid0)
            drain_gather(rows0, self0, sem_a)
            reduce_write(w0, rows0, self0)
            wait_idx(nod0, nid0)
            fire_gather(nod0, nid0, rows0, self0, sem_a)

            @pl.when(w3 < n_windows)
            def _():
                fire_idx(w3, nod1, nid1)

            drain_gather(rows1, self1, sem_b)
            reduce_write(w0 + 1, rows1, self1)
            return carry

        lax.fori_loop(0, (n_windows - 1) // 2, pair_body, 0)
        drain_gather(rows0, self0, sem_a)
        reduce_write(n_windows - 1, rows0, self0)

    return sc_kernel


def _tc_matmul_kernel(w1_ref, w2_ref, x1_ref, x2_ref, o_ref):
    a = lax.dot_general(
        w1_ref[...], x1_ref[...], (((1,), (1,)), ((), ())),
        preferred_element_type=jnp.float32,
    )
    b = lax.dot_general(
        w2_ref[...], x2_ref[...], (((1,), (1,)), ((), ())),
        preferred_element_type=jnp.float32,
    )
    o_ref[...] = jnp.maximum(a + b, 0.0)


def kernel(feat_table, W, nodes, neigh_idx):
    n_nodes, feat_dim = feat_table.shape
    batch = nodes.shape[0]
    num_sample = neigh_idx.shape[1]
    embed_dim = W.shape[0]

    quantum = _NW * _WIN
    b_pad = ((batch + quantum - 1) // quantum) * quantum
    if (b_pad // quantum) % 2 == 0:
        b_pad += quantum
    n_per_worker = b_pad // _NW
    n_windows = n_per_worker // _WIN

    nodes_p = jnp.pad(nodes.astype(jnp.int32), (0, b_pad - batch))
    narr = (
        jnp.pad(neigh_idx.astype(jnp.int32), ((0, b_pad - batch), (0, 0)))
        .reshape(_NW * n_windows, _WIN, num_sample)
        .transpose(0, 2, 1)
        .reshape(-1)
    )

    tab_w = jax.lax.bitcast_convert_type(
        feat_table.astype(jnp.bfloat16).reshape(n_nodes, feat_dim // 2, 2),
        jnp.int32).reshape(-1)
    sc = _sc_gather(n_per_worker, n_windows, num_sample, feat_dim, b_pad)
    self_w, sum_w = sc(nodes_p, narr, tab_w)

    def _unpack(x):
        return jax.lax.bitcast_convert_type(
            x.reshape(b_pad, feat_dim // 2), jnp.bfloat16
        ).reshape(b_pad, feat_dim)

    self_feats = _unpack(self_w)
    neigh_sum = _unpack(sum_w)

    w1 = W[:, :feat_dim].astype(jnp.bfloat16)
    w2 = (W[:, feat_dim:] * (1.0 / num_sample)).astype(jnp.bfloat16)

    bn = 512
    grid = (b_pad // bn,)
    out = pl.pallas_call(
        _tc_matmul_kernel,
        grid=grid,
        in_specs=[
            pl.BlockSpec((embed_dim, feat_dim), lambda i: (0, 0)),
            pl.BlockSpec((embed_dim, feat_dim), lambda i: (0, 0)),
            pl.BlockSpec((bn, feat_dim), lambda i: (i, 0)),
            pl.BlockSpec((bn, feat_dim), lambda i: (i, 0)),
        ],
        cost_estimate=pl.CostEstimate(
            flops=2 * 2 * b_pad * feat_dim * embed_dim,
            bytes_accessed=4 * b_pad * feat_dim + 4 * embed_dim * b_pad,
            transcendentals=0,
        ),
        out_specs=pl.BlockSpec((embed_dim, bn), lambda i: (0, i)),
        out_shape=jax.ShapeDtypeStruct((embed_dim, b_pad), jnp.float32),
    )(w1, w2, self_feats, neigh_sum)

    return out[:, :batch]


# speedup vs baseline: 1.9796x; 1.9796x over previous
import functools

import jax
import jax.numpy as jnp
from jax import lax
from jax.experimental import pallas as pl
from jax.experimental.pallas import tpu as pltpu
from jax.experimental.pallas import tpu_sc as plsc

_NC = 2
_NS = 16
_NW = _NC * _NS

_WIN = 32
_L = 16


def _sc_gather(n_per_worker, n_windows, num_sample, feat_dim, b_pad):
    mesh = plsc.VectorSubcoreMesh(core_axis_name="c", subcore_axis_name="s")
    rowsz = feat_dim // 2
    nbuf_sz = _WIN * num_sample * rowsz
    obuf_sz = _WIN * rowsz

    assert n_windows % 2 == 1 and n_windows >= 3

    @functools.partial(
        pl.kernel,
        out_type=(
            jax.ShapeDtypeStruct((b_pad * rowsz,), jnp.int32),
            jax.ShapeDtypeStruct((b_pad * rowsz,), jnp.int32),
        ),
        mesh=mesh,
        scratch_types=[
            pltpu.VMEM((_WIN,), jnp.int32),
            pltpu.VMEM((_WIN,), jnp.int32),
            pltpu.VMEM((num_sample * _WIN,), jnp.int32),
            pltpu.VMEM((num_sample * _WIN,), jnp.int32),
            pltpu.VMEM((nbuf_sz,), jnp.int32),
            pltpu.VMEM((nbuf_sz,), jnp.int32),
            pltpu.VMEM((obuf_sz,), jnp.int32),
            pltpu.VMEM((obuf_sz,), jnp.int32),
            pltpu.VMEM((obuf_sz,), jnp.int32),
            pltpu.SemaphoreType.DMA,
            pltpu.SemaphoreType.DMA,
            pltpu.SemaphoreType.DMA,
        ],
    )
    def sc_kernel(nodes_hbm, narr_hbm, tab_hbm, self_hbm, sum_hbm,
                  nod0, nod1, nid0, nid1, rows0, rows1, self0, self1, acc_v,
                  sem_a, sem_b, sem_i):
        wid = lax.axis_index("s") * _NC + lax.axis_index("c")
        base = wid * n_per_worker

        def fire_idx(w, nod_iv, nid_iv):
            pltpu.async_copy(
                nodes_hbm.at[pl.ds(base + w * _WIN, _WIN)], nod_iv, sem_i)
            pltpu.async_copy(
                narr_hbm.at[pl.ds((wid * n_windows + w) * num_sample * _WIN,
                                  num_sample * _WIN)],
                nid_iv, sem_i)

        def wait_idx(nod_iv, nid_iv):
            pltpu.make_async_copy(
                nodes_hbm.at[pl.ds(0, _WIN)], nod_iv, sem_i).wait()
            pltpu.make_async_copy(
                narr_hbm.at[pl.ds(0, num_sample * _WIN)], nid_iv, sem_i).wait()

        def fire_gather(nod_iv, nid_iv, nrows_v, self_v, sem):
            def group_fire(g, carry):
                nv = nod_iv[pl.ds(g * _L, _L)]
                for i in range(_L):
                    s = nv[i] * rowsz
                    pltpu.async_copy(
                        tab_hbm.at[pl.ds(s, rowsz)],
                        self_v.at[pl.ds((g * _L + i) * rowsz, rowsz)], sem)
                for j in range(num_sample):
                    tv = nid_iv[pl.ds(j * _WIN + g * _L, _L)]
                    for i in range(_L):
                        t = tv[i] * rowsz
                        pltpu.async_copy(
                            tab_hbm.at[pl.ds(t, rowsz)],
                            nrows_v.at[pl.ds(
                                ((g * _L + i) * num_sample + j) * rowsz,
                                rowsz)],
                            sem)
                return carry

            lax.fori_loop(0, _WIN // _L, group_fire, 0)

        def drain_gather(nrows_v, self_v, sem):
            pltpu.make_async_copy(
                tab_hbm.at[pl.ds(0, nbuf_sz)], nrows_v, sem).wait()
            pltpu.make_async_copy(
                tab_hbm.at[pl.ds(0, obuf_sz)], self_v, sem).wait()

        def reduce_write(w, nrows_v, self_v):
            def node_red(i, carry):
                noff = i * num_sample * rowsz
                for c in range(rowsz // _L):
                    off = noff + c * _L
                    w = nrows_v[pl.ds(off, _L)]
                    lo = lax.bitcast_convert_type(w << 16, jnp.float32)
                    hi = lax.bitcast_convert_type(w & -65536, jnp.float32)
                    for j in range(1, num_sample):
                        w = nrows_v[pl.ds(off + j * rowsz, _L)]
                        lo = lo + lax.bitcast_convert_type(w << 16, jnp.float32)
                        hi = hi + lax.bitcast_convert_type(w & -65536, jnp.float32)
                    lo_w = lax.shift_right_logical(
                        lax.bitcast_convert_type(lo, jnp.int32) + 32768, 16)
                    hi_w = (lax.bitcast_convert_type(hi, jnp.int32) + 32768) & -65536
                    acc_v[pl.ds(i * rowsz + c * _L, _L)] = lo_w | hi_w
                return carry

            lax.fori_loop(0, _WIN, node_red, 0)
            row0 = base + w * _WIN
            pltpu.sync_copy(self_v, self_hbm.at[pl.ds(row0 * rowsz, obuf_sz)])
            pltpu.sync_copy(acc_v, sum_hbm.at[pl.ds(row0 * rowsz, obuf_sz)])

        fire_idx(0, nod0, nid0)
        wait_idx(nod0, nid0)
        fire_gather(nod0, nid0, rows0, self0, sem_a)
        fire_idx(1, nod1, nid1)

        def pair_body(p, carry):
            w0 = 2 * p
            w3 = w0 + 3
            wait_idx(nod1, nid1)
            fire_gather(nod1, nid1, rows1, self1, sem_b)
            fire_idx(w0 + 2, nod0, nid0)
            drain_gather(rows0, self0, sem_a)
            reduce_write(w0, rows0, self0)
            wait_idx(nod0, nid0)
            fire_gather(nod0, nid0, rows0, self0, sem_a)

            @pl.when(w3 < n_windows)
            def _():
                fire_idx(w3, nod1, nid1)

            drain_gather(rows1, self1, sem_b)
            reduce_write(w0 + 1, rows1, self1)
            return carry

        lax.fori_loop(0, (n_windows - 1) // 2, pair_body, 0)
        drain_gather(rows0, self0, sem_a)
        reduce_write(n_windows - 1, rows0, self0)

    return sc_kernel


def _tc_pack_kernel(x_ref, o_ref):
    half = x_ref.shape[1] // 2
    lo = lax.bitcast_convert_type(
        x_ref[:, :half].astype(jnp.bfloat16), jnp.uint16).astype(jnp.uint32)
    hi = lax.bitcast_convert_type(
        x_ref[:, half:].astype(jnp.bfloat16), jnp.uint16).astype(jnp.uint32)
    o_ref[...] = lax.bitcast_convert_type(lo | (hi << 16), jnp.int32)


def _tc_matmul_kernel(w1a_ref, w1b_ref, w2a_ref, w2b_ref, xs_ref, xn_ref,
                      o_ref):
    def halves(w):
        return (lax.bitcast_convert_type(w << 16, jnp.float32),
                lax.bitcast_convert_type(w & -65536, jnp.float32))

    dn = (((1,), (1,)), ((), ()))
    xsl, xsh = halves(xs_ref[...])
    xnl, xnh = halves(xn_ref[...])
    acc = lax.dot_general(w1a_ref[...], xsl, dn,
                          preferred_element_type=jnp.float32)
    acc += lax.dot_general(w1b_ref[...], xsh, dn,
                           preferred_element_type=jnp.float32)
    acc += lax.dot_general(w2a_ref[...], xnl, dn,
                           preferred_element_type=jnp.float32)
    acc += lax.dot_general(w2b_ref[...], xnh, dn,
                           preferred_element_type=jnp.float32)
    o_ref[...] = jnp.maximum(acc, 0.0)


def kernel(feat_table, W, nodes, neigh_idx):
    n_nodes, feat_dim = feat_table.shape
    batch = nodes.shape[0]
    num_sample = neigh_idx.shape[1]
    embed_dim = W.shape[0]

    quantum = _NW * _WIN
    b_pad = ((batch + quantum - 1) // quantum) * quantum
    if (b_pad // quantum) % 2 == 0:
        b_pad += quantum
    n_per_worker = b_pad // _NW
    n_windows = n_per_worker // _WIN

    nodes_p = jnp.pad(nodes.astype(jnp.int32), (0, b_pad - batch))
    narr = (
        jnp.pad(neigh_idx.astype(jnp.int32), ((0, b_pad - batch), (0, 0)))
        .reshape(_NW * n_windows, _WIN, num_sample)
        .transpose(0, 2, 1)
        .reshape(-1)
    )

    words = feat_dim // 2
    bv = 512
    tab_w = pl.pallas_call(
        _tc_pack_kernel,
        grid=((n_nodes + bv - 1) // bv,),
        in_specs=[pl.BlockSpec((bv, feat_dim), lambda i: (i, 0))],
        out_specs=pl.BlockSpec((bv, words), lambda i: (i, 0)),
        out_shape=jax.ShapeDtypeStruct((n_nodes, words), jnp.int32),
    )(feat_table).reshape(-1)

    sc = _sc_gather(n_per_worker, n_windows, num_sample, feat_dim, b_pad)
    self_w, sum_w = sc(nodes_p, narr, tab_w)
    self_w = self_w.reshape(b_pad, words)
    sum_w = sum_w.reshape(b_pad, words)

    w1 = W[:, :feat_dim]
    w2 = W[:, feat_dim:] * (1.0 / num_sample)
    w1a, w1b = w1[:, :words], w1[:, words:]
    w2a, w2b = w2[:, :words], w2[:, words:]

    bn = 512
    grid = ((batch + bn - 1) // bn,)
    out = pl.pallas_call(
        _tc_matmul_kernel,
        grid=grid,
        in_specs=[
            pl.BlockSpec((embed_dim, words), lambda i: (0, 0)),
            pl.BlockSpec((embed_dim, words), lambda i: (0, 0)),
            pl.BlockSpec((embed_dim, words), lambda i: (0, 0)),
            pl.BlockSpec((embed_dim, words), lambda i: (0, 0)),
            pl.BlockSpec((bn, words), lambda i: (i, 0)),
            pl.BlockSpec((bn, words), lambda i: (i, 0)),
        ],
        out_specs=pl.BlockSpec((embed_dim, bn), lambda i: (0, i)),
        out_shape=jax.ShapeDtypeStruct((embed_dim, batch), jnp.float32),
    )(w1a, w1b, w2a, w2b, self_w, sum_w)

    return out


# speedup vs baseline: 2.7066x; 1.3672x over previous
import functools

import jax
import jax.numpy as jnp
from jax import lax
from jax.experimental import pallas as pl
from jax.experimental.pallas import tpu as pltpu
from jax.experimental.pallas import tpu_sc as plsc

_NC = 2
_NS = 16
_NW = _NC * _NS

_WIN = 32
_L = 16


def _sc_gather(n_per_worker, n_windows, num_sample, feat_dim, b_pad):
    mesh = plsc.VectorSubcoreMesh(core_axis_name="c", subcore_axis_name="s")
    rowsz = feat_dim // 2
    nbuf_sz = _WIN * num_sample * rowsz
    obuf_sz = _WIN * rowsz

    assert n_windows % 2 == 1 and n_windows >= 3

    @functools.partial(
        pl.kernel,
        out_type=jax.ShapeDtypeStruct((b_pad * 2 * rowsz,), jnp.int32),
        mesh=mesh,
        scratch_types=[
            pltpu.VMEM((_WIN,), jnp.int32),
            pltpu.VMEM((_WIN,), jnp.int32),
            pltpu.VMEM((num_sample * _WIN,), jnp.int32),
            pltpu.VMEM((num_sample * _WIN,), jnp.int32),
            pltpu.VMEM((nbuf_sz,), jnp.int32),
            pltpu.VMEM((nbuf_sz,), jnp.int32),
            pltpu.VMEM((2 * obuf_sz,), jnp.int32),
            pltpu.VMEM((2 * obuf_sz,), jnp.int32),
            pltpu.SemaphoreType.DMA,
            pltpu.SemaphoreType.DMA,
            pltpu.SemaphoreType.DMA,
        ],
    )
    def sc_kernel(nodes_hbm, narr_hbm, tab_hbm, comb_hbm,
                  nod0, nod1, nid0, nid1, rows0, rows1, comb0, comb1,
                  sem_a, sem_b, sem_i):
        wid = lax.axis_index("s") * _NC + lax.axis_index("c")
        base = wid * n_per_worker

        def fire_idx(w, nod_iv, nid_iv):
            pltpu.async_copy(
                nodes_hbm.at[pl.ds(base + w * _WIN, _WIN)], nod_iv, sem_i)
            pltpu.async_copy(
                narr_hbm.at[pl.ds((wid * n_windows + w) * num_sample * _WIN,
                                  num_sample * _WIN)],
                nid_iv, sem_i)

        def wait_idx(nod_iv, nid_iv):
            pltpu.make_async_copy(
                nodes_hbm.at[pl.ds(0, _WIN)], nod_iv, sem_i).wait()
            pltpu.make_async_copy(
                narr_hbm.at[pl.ds(0, num_sample * _WIN)], nid_iv, sem_i).wait()

        def woff(v):
            return (((v & -1024) << 6) + ((v & 511) << 7)
                    + lax.shift_right_logical(v & 512, 3))

        def fire_gather(nod_iv, nid_iv, nrows_v, comb_v, sem):
            def group_fire(g, carry):
                nv = woff(nod_iv[pl.ds(g * _L, _L)])
                for i in range(_L):
                    pltpu.async_copy(
                        tab_hbm.at[pl.ds(pl.multiple_of(nv[i], 64), rowsz)],
                        comb_v.at[pl.ds((g * _L + i) * 2 * rowsz, rowsz)],
                        sem)
                for j in range(num_sample):
                    tv = woff(nid_iv[pl.ds(j * _WIN + g * _L, _L)])
                    for i in range(_L):
                        pltpu.async_copy(
                            tab_hbm.at[pl.ds(pl.multiple_of(tv[i], 64),
                                             rowsz)],
                            nrows_v.at[pl.ds(
                                ((g * _L + i) * num_sample + j) * rowsz,
                                rowsz)],
                            sem)
                return carry

            lax.fori_loop(0, _WIN // _L, group_fire, 0)

        def drain_gather(nrows_v, comb_v, sem):
            pltpu.make_async_copy(
                tab_hbm.at[pl.ds(0, nbuf_sz)], nrows_v, sem).wait()
            pltpu.make_async_copy(
                tab_hbm.at[pl.ds(0, obuf_sz)], comb_v.at[pl.ds(0, obuf_sz)],
                sem).wait()

        def reduce_write(w, nrows_v, comb_v):
            def node_red(i, carry):
                noff = i * num_sample * rowsz
                for c in range(rowsz // _L):
                    off = noff + c * _L
                    w = nrows_v[pl.ds(off, _L)]
                    lo = lax.bitcast_convert_type(w << 16, jnp.float32)
                    hi = lax.bitcast_convert_type(w & -65536, jnp.float32)
                    for j in range(1, num_sample):
                        w = nrows_v[pl.ds(off + j * rowsz, _L)]
                        lo = lo + lax.bitcast_convert_type(w << 16, jnp.float32)
                        hi = hi + lax.bitcast_convert_type(w & -65536, jnp.float32)
                    lo_w = lax.shift_right_logical(
                        lax.bitcast_convert_type(lo, jnp.int32) + 32768, 16)
                    hi_w = (lax.bitcast_convert_type(hi, jnp.int32) + 32768) & -65536
                    comb_v[pl.ds((2 * i + 1) * rowsz + c * _L, _L)] = lo_w | hi_w
                return carry

            lax.fori_loop(0, _WIN, node_red, 0)
            row0 = base + w * _WIN
            pltpu.sync_copy(
                comb_v, comb_hbm.at[pl.ds(row0 * 2 * rowsz, 2 * obuf_sz)])

        fire_idx(0, nod0, nid0)
        wait_idx(nod0, nid0)
        fire_gather(nod0, nid0, rows0, comb0, sem_a)
        fire_idx(1, nod1, nid1)

        def pair_body(p, carry):
            w0 = 2 * p
            w3 = w0 + 3
            wait_idx(nod1, nid1)
            fire_gather(nod1, nid1, rows1, comb1, sem_b)
            fire_idx(w0 + 2, nod0, nid0)
            drain_gather(rows0, comb0, sem_a)
            reduce_write(w0, rows0, comb0)
            wait_idx(nod0, nid0)
            fire_gather(nod0, nid0, rows0, comb0, sem_a)

            @pl.when(w3 < n_windows)
            def _():
                fire_idx(w3, nod1, nid1)

            drain_gather(rows1, comb1, sem_b)
            reduce_write(w0 + 1, rows1, comb1)
            return carry

        lax.fori_loop(0, (n_windows - 1) // 2, pair_body, 0)
        drain_gather(rows0, comb0, sem_a)
        reduce_write(n_windows - 1, rows0, comb0)

    return sc_kernel


def _tc_pack_kernel(xa_ref, xb_ref, o_ref):
    half = xa_ref.shape[1] // 2

    def pack(x):
        lo = lax.bitcast_convert_type(
            x[:, :half].astype(jnp.bfloat16), jnp.uint16).astype(jnp.uint32)
        hi = lax.bitcast_convert_type(
            x[:, half:].astype(jnp.bfloat16), jnp.uint16).astype(jnp.uint32)
        return lax.bitcast_convert_type(lo | (hi << 16), jnp.int32)

    o_ref[:, :half] = pack(xa_ref[...])
    o_ref[:, half:] = pack(xb_ref[...])


def _tc_matmul_kernel(wlo_ref, whi_ref, x_ref, o_ref):
    dn = (((1,), (1,)), ((), ()))
    x = x_ref[...]
    xlo = lax.bitcast_convert_type(x << 16, jnp.float32)
    xhi = lax.bitcast_convert_type(x & -65536, jnp.float32)
    acc = lax.dot_general(wlo_ref[...], xlo, dn,
                          preferred_element_type=jnp.float32)
    acc += lax.dot_general(whi_ref[...], xhi, dn,
                           preferred_element_type=jnp.float32)
    o_ref[...] = jnp.maximum(acc, 0.0)


def kernel(feat_table, W, nodes, neigh_idx):
    n_nodes, feat_dim = feat_table.shape
    batch = nodes.shape[0]
    num_sample = neigh_idx.shape[1]
    embed_dim = W.shape[0]

    quantum = _NW * _WIN
    b_pad = ((batch + quantum - 1) // quantum) * quantum
    if (b_pad // quantum) % 2 == 0:
        b_pad += quantum
    n_per_worker = b_pad // _NW
    n_windows = n_per_worker // _WIN

    nodes_p = jnp.pad(nodes.astype(jnp.int32), (0, b_pad - batch))
    narr = (
        jnp.pad(neigh_idx.astype(jnp.int32), ((0, b_pad - batch), (0, 0)))
        .reshape(_NW * n_windows, _WIN, num_sample)
        .transpose(0, 2, 1)
        .reshape(-1)
    )

    words = feat_dim // 2
    bv = 1024
    bvh = bv // 2
    n_pack = (n_nodes + bv - 1) // bv
    tab_w = pl.pallas_call(
        _tc_pack_kernel,
        grid=(n_pack,),
        in_specs=[
            pl.BlockSpec((bvh, feat_dim), lambda i: (2 * i, 0)),
            pl.BlockSpec((bvh, feat_dim), lambda i: (2 * i + 1, 0)),
        ],
        out_specs=pl.BlockSpec((bvh, feat_dim), lambda i: (i, 0)),
        out_shape=jax.ShapeDtypeStruct((n_pack * bvh, feat_dim), jnp.int32),
    )(feat_table, feat_table).reshape(-1)

    sc = _sc_gather(n_per_worker, n_windows, num_sample, feat_dim, b_pad)
    comb = sc(nodes_p, narr, tab_w).reshape(b_pad, feat_dim)

    w1 = W[:, :feat_dim]
    w2 = W[:, feat_dim:] * (1.0 / num_sample)
    wlo = jnp.concatenate([w1[:, :words], w2[:, :words]], axis=1)
    whi = jnp.concatenate([w1[:, words:], w2[:, words:]], axis=1)

    bn = 512
    grid = ((batch + bn - 1) // bn,)
    out = pl.pallas_call(
        _tc_matmul_kernel,
        grid=grid,
        in_specs=[
            pl.BlockSpec((embed_dim, feat_dim), lambda i: (0, 0)),
            pl.BlockSpec((embed_dim, feat_dim), lambda i: (0, 0)),
            pl.BlockSpec((bn, feat_dim), lambda i: (i, 0)),
        ],
        out_specs=pl.BlockSpec((embed_dim, bn), lambda i: (0, i)),
        out_shape=jax.ShapeDtypeStruct((embed_dim, batch), jnp.float32),
    )(wlo, whi, comb)

    return out


# speedup vs baseline: 2.9906x; 1.1049x over previous
import functools

import jax
import jax.numpy as jnp
from jax import lax
from jax.experimental import pallas as pl
from jax.experimental.pallas import tpu as pltpu
from jax.experimental.pallas import tpu_sc as plsc

_NC = 2
_NS = 16
_NW = _NC * _NS

_WIN = 32
_L = 16


def _sc_gather(n_per_worker, n_windows, num_sample, feat_dim, b_pad):
    mesh = plsc.VectorSubcoreMesh(core_axis_name="c", subcore_axis_name="s")
    rowsz = feat_dim // 2
    nbuf_sz = _WIN * num_sample * rowsz
    obuf_sz = _WIN * rowsz

    assert n_windows % 2 == 1 and n_windows >= 3

    @functools.partial(
        pl.kernel,
        out_type=jax.ShapeDtypeStruct((b_pad * 2 * rowsz,), jnp.int32),
        mesh=mesh,
        scratch_types=[
            pltpu.VMEM((_WIN,), jnp.int32),
            pltpu.VMEM((_WIN,), jnp.int32),
            pltpu.VMEM((num_sample * _WIN,), jnp.int32),
            pltpu.VMEM((num_sample * _WIN,), jnp.int32),
            pltpu.VMEM((nbuf_sz,), jnp.int32),
            pltpu.VMEM((nbuf_sz,), jnp.int32),
            pltpu.VMEM((2 * obuf_sz,), jnp.int32),
            pltpu.VMEM((2 * obuf_sz,), jnp.int32),
            pltpu.SemaphoreType.DMA,
            pltpu.SemaphoreType.DMA,
            pltpu.SemaphoreType.DMA,
        ],
    )
    def sc_kernel(nodes_hbm, narr_hbm, tab_hbm, comb_hbm,
                  nod0, nod1, nid0, nid1, rows0, rows1, comb0, comb1,
                  sem_a, sem_b, sem_i):
        wid = lax.axis_index("s") * _NC + lax.axis_index("c")
        base = wid * n_per_worker

        def fire_idx(w, nod_iv, nid_iv):
            pltpu.async_copy(
                nodes_hbm.at[pl.ds(base + w * _WIN, _WIN)], nod_iv, sem_i)
            pltpu.async_copy(
                narr_hbm.at[pl.ds((wid * n_windows + w) * num_sample * _WIN,
                                  num_sample * _WIN)],
                nid_iv, sem_i)

        def wait_idx(nod_iv, nid_iv):
            pltpu.make_async_copy(
                nodes_hbm.at[pl.ds(0, _WIN)], nod_iv, sem_i).wait()
            pltpu.make_async_copy(
                narr_hbm.at[pl.ds(0, num_sample * _WIN)], nid_iv, sem_i).wait()

        def woff(v):
            return (((v & -1024) << 6) + ((v & 511) << 7)
                    + lax.shift_right_logical(v & 512, 3))

        def fire_gather(nod_iv, nid_iv, nrows_v, comb_v, sem):
            def group_fire(g, carry):
                nv = woff(nod_iv[pl.ds(g * _L, _L)])
                for i in range(_L):
                    pltpu.async_copy(
                        tab_hbm.at[pl.ds(pl.multiple_of(nv[i], 64), rowsz)],
                        comb_v.at[pl.ds((g * _L + i) * 2 * rowsz, rowsz)],
                        sem)
                for j in range(num_sample):
                    tv = woff(nid_iv[pl.ds(j * _WIN + g * _L, _L)])
                    for i in range(_L):
                        pltpu.async_copy(
                            tab_hbm.at[pl.ds(pl.multiple_of(tv[i], 64),
                                             rowsz)],
                            nrows_v.at[pl.ds(
                                ((g * _L + i) * num_sample + j) * rowsz,
                                rowsz)],
                            sem)
                return carry

            lax.fori_loop(0, _WIN // _L, group_fire, 0)

        def drain_gather(nrows_v, comb_v, sem):
            pltpu.make_async_copy(
                tab_hbm.at[pl.ds(0, nbuf_sz)], nrows_v, sem).wait()
            pltpu.make_async_copy(
                tab_hbm.at[pl.ds(0, obuf_sz)], comb_v.at[pl.ds(0, obuf_sz)],
                sem).wait()

        def reduce_write(w, nrows_v, comb_v):
            def node_red(i, carry):
                noff = i * num_sample * rowsz
                for c in range(rowsz // _L):
                    off = noff + c * _L
                    w = nrows_v[pl.ds(off, _L)]
                    lo = lax.bitcast_convert_type(w << 16, jnp.float32)
                    hi = lax.bitcast_convert_type(w & -65536, jnp.float32)
                    for j in range(1, num_sample):
                        w = nrows_v[pl.ds(off + j * rowsz, _L)]
                        lo = lo + lax.bitcast_convert_type(w << 16, jnp.float32)
                        hi = hi + lax.bitcast_convert_type(w & -65536, jnp.float32)
                    lo_w = lax.shift_right_logical(
                        lax.bitcast_convert_type(lo, jnp.int32) + 32768, 16)
                    hi_w = (lax.bitcast_convert_type(hi, jnp.int32) + 32768) & -65536
                    comb_v[pl.ds((2 * i + 1) * rowsz + c * _L, _L)] = lo_w | hi_w
                return carry

            lax.fori_loop(0, _WIN, node_red, 0)
            row0 = base + w * _WIN
            pltpu.sync_copy(
                comb_v, comb_hbm.at[pl.ds(row0 * 2 * rowsz, 2 * obuf_sz)])

        fire_idx(0, nod0, nid0)
        wait_idx(nod0, nid0)
        fire_gather(nod0, nid0, rows0, comb0, sem_a)
        fire_idx(1, nod1, nid1)

        def pair_body(p, carry):
            w0 = 2 * p
            w3 = w0 + 3
            wait_idx(nod1, nid1)
            fire_gather(nod1, nid1, rows1, comb1, sem_b)
            fire_idx(w0 + 2, nod0, nid0)
            drain_gather(rows0, comb0, sem_a)
            reduce_write(w0, rows0, comb0)
            wait_idx(nod0, nid0)
            fire_gather(nod0, nid0, rows0, comb0, sem_a)

            @pl.when(w3 < n_windows)
            def _():
                fire_idx(w3, nod1, nid1)

            drain_gather(rows1, comb1, sem_b)
            reduce_write(w0 + 1, rows1, comb1)
            return carry

        lax.fori_loop(0, (n_windows - 1) // 2, pair_body, 0)
        drain_gather(rows0, comb0, sem_a)
        reduce_write(n_windows - 1, rows0, comb0)

    return sc_kernel


def _tc_pack_kernel(xa_ref, xb_ref, o_ref):
    half = xa_ref.shape[1] // 2

    def pack(x):
        lo = lax.bitcast_convert_type(
            x[:, :half].astype(jnp.bfloat16), jnp.uint16).astype(jnp.uint32)
        hi = lax.bitcast_convert_type(
            x[:, half:].astype(jnp.bfloat16), jnp.uint16).astype(jnp.uint32)
        return lax.bitcast_convert_type(lo | (hi << 16), jnp.int32)

    o_ref[:, :half] = pack(xa_ref[...])
    o_ref[:, half:] = pack(xb_ref[...])


def _tc_matmul_kernel(wlo_ref, whi_ref, x_ref, o_ref):
    dn = (((1,), (1,)), ((), ()))
    x = x_ref[...]
    xlo = lax.bitcast_convert_type(x << 16, jnp.float32)
    xhi = lax.bitcast_convert_type(x & -65536, jnp.float32)
    acc = lax.dot_general(wlo_ref[...], xlo, dn,
                          preferred_element_type=jnp.float32)
    acc += lax.dot_general(whi_ref[...], xhi, dn,
                           preferred_element_type=jnp.float32)
    o_ref[...] = jnp.maximum(acc, 0.0)


def kernel(feat_table, W, nodes, neigh_idx):
    n_nodes, feat_dim = feat_table.shape
    batch = nodes.shape[0]
    num_sample = neigh_idx.shape[1]
    embed_dim = W.shape[0]

    quantum = _NW * _WIN
    b_pad = ((batch + quantum - 1) // quantum) * quantum
    if (b_pad // quantum) % 2 == 0:
        b_pad += quantum
    n_per_worker = b_pad // _NW
    n_windows = n_per_worker // _WIN

    nodes_p = jnp.pad(nodes.astype(jnp.int32), (0, b_pad - batch))
    narr = (
        jnp.pad(neigh_idx.astype(jnp.int32), ((0, b_pad - batch), (0, 0)))
        .reshape(_NW * n_windows, _WIN, num_sample)
        .transpose(0, 2, 1)
        .reshape(-1)
    )

    words = feat_dim // 2
    bv = 1024
    bvh = bv // 2
    n_pack = (n_nodes + bv - 1) // bv
    tab_w = pl.pallas_call(
        _tc_pack_kernel,
        grid=(n_pack,),
        in_specs=[
            pl.BlockSpec((bvh, feat_dim), lambda i: (2 * i, 0)),
            pl.BlockSpec((bvh, feat_dim), lambda i: (2 * i + 1, 0)),
        ],
        out_specs=pl.BlockSpec((bvh, feat_dim), lambda i: (i, 0)),
        out_shape=jax.ShapeDtypeStruct((n_pack * bvh, feat_dim), jnp.int32),
    )(feat_table, feat_table).reshape(-1)

    sc = _sc_gather(n_per_worker, n_windows, num_sample, feat_dim, b_pad)
    comb = sc(nodes_p, narr, tab_w).reshape(b_pad, feat_dim)

    w1 = W[:, :feat_dim]
    w2 = W[:, feat_dim:] * (1.0 / num_sample)
    wlo = jnp.concatenate([w1[:, :words], w2[:, :words]], axis=1)
    whi = jnp.concatenate([w1[:, words:], w2[:, words:]], axis=1)

    bn = 2048
    grid = ((batch + bn - 1) // bn,)
    out = pl.pallas_call(
        _tc_matmul_kernel,
        grid=grid,
        in_specs=[
            pl.BlockSpec((embed_dim, feat_dim), lambda i: (0, 0)),
            pl.BlockSpec((embed_dim, feat_dim), lambda i: (0, 0)),
            pl.BlockSpec((bn, feat_dim), lambda i: (i, 0)),
        ],
        out_specs=pl.BlockSpec((embed_dim, bn), lambda i: (0, i)),
        out_shape=jax.ShapeDtypeStruct((embed_dim, batch), jnp.float32),
    )(wlo, whi, comb)

    return out
